# baseline (speedup 1.0000x reference)
import functools

import jax
import jax.numpy as jnp
from jax import lax
from jax.experimental import pallas as pl
from jax.experimental.pallas import tpu as pltpu
from jax.experimental.pallas import tpu_sc as plsc

INPUT_DIM = 2048
SIDE_DIM = 2048
CONTEXT_DIM = 4
NUM_CTX = 2 ** CONTEXT_DIM
BATCH = 4096
BB = 512

NC = 2
NS = 16
NW = NC * NS
BPW = BATCH // NW
LANES = 16


def _tc_body(side_ref, logit_ref, v_ref, b_ref, w_ref, bc_ref, ctx_ref, dots_ref):
    proj = jnp.dot(v_ref[...], side_ref[...],
                   preferred_element_type=jnp.float32)
    bits = (proj > b_ref[...]).astype(jnp.float32)
    ctx_ref[0, :] = jnp.sum(bits * bc_ref[...], axis=0).astype(jnp.int32)
    dots = jnp.dot(w_ref[...], logit_ref[...],
                   preferred_element_type=jnp.float32)
    wpb = BB // BPW
    dots_ref[...] = (
        dots.reshape(NUM_CTX, wpb, BPW).swapaxes(0, 1).reshape(wpb, 1, NUM_CTX * BPW)
    )


def _sc_route(ctx_hbm, dots_hbm, out_hbm, ctx_v, dots_v, out_v):
    wid = lax.axis_index("s") * NC + lax.axis_index("c")
    base = wid * BPW
    pltpu.sync_copy(ctx_hbm.at[pl.ds(base, BPW)], ctx_v)
    pltpu.sync_copy(dots_hbm.at[wid, 0], dots_v)
    for i in range(BPW // LANES):
        rows = ctx_v[pl.ds(i * LANES, LANES)]
        acc = jnp.zeros((LANES,), jnp.float32)
        for k in range(NUM_CTX):
            val = dots_v[pl.ds(k * BPW + i * LANES, LANES)]
            acc = jnp.where(rows == k, val, acc)
        out_v[pl.ds(i * LANES, LANES)] = acc
    pltpu.sync_copy(out_v, out_hbm.at[pl.ds(base, BPW)])


def kernel(logit_previous, side_information, v, b, weights, boolean_converter):
    grid = BATCH // BB
    ctx2d, dots = pl.pallas_call(
        _tc_body,
        grid=(grid,),
        in_specs=[
            pl.BlockSpec((SIDE_DIM, BB), lambda i: (0, i)),
            pl.BlockSpec((INPUT_DIM, BB), lambda i: (0, i)),
            pl.BlockSpec((CONTEXT_DIM, SIDE_DIM), lambda i: (0, 0)),
            pl.BlockSpec((CONTEXT_DIM, 1), lambda i: (0, 0)),
            pl.BlockSpec((NUM_CTX, INPUT_DIM), lambda i: (0, 0)),
            pl.BlockSpec((CONTEXT_DIM, 1), lambda i: (0, 0)),
        ],
        out_specs=[
            pl.BlockSpec((1, BB), lambda i: (0, i)),
            pl.BlockSpec((BB // BPW, 1, NUM_CTX * BPW), lambda i: (i, 0, 0)),
        ],
        out_shape=[
            jax.ShapeDtypeStruct((1, BATCH), jnp.int32),
            jax.ShapeDtypeStruct((NW, 1, NUM_CTX * BPW), jnp.float32),
        ],
    )(side_information, logit_previous, v, b, weights, boolean_converter)

    route = functools.partial(
        pl.kernel,
        mesh=plsc.VectorSubcoreMesh(core_axis_name="c", subcore_axis_name="s"),
        out_type=jax.ShapeDtypeStruct((BATCH,), jnp.float32),
        scratch_types=[
            pltpu.VMEM((BPW,), jnp.int32),
            pltpu.VMEM((NUM_CTX * BPW,), jnp.float32),
            pltpu.VMEM((BPW,), jnp.float32),
        ],
    )(_sc_route)
    return route(ctx2d.reshape(BATCH), dots)


# speedup vs baseline: 1.0106x; 1.0106x over previous
import functools

import jax
import jax.numpy as jnp
from jax import lax
from jax.experimental import pallas as pl
from jax.experimental.pallas import tpu as pltpu
from jax.experimental.pallas import tpu_sc as plsc

INPUT_DIM = 2048
SIDE_DIM = 2048
CONTEXT_DIM = 4
NUM_CTX = 2 ** CONTEXT_DIM
BATCH = 4096
BB = 512

NC = 2
NS = 16
NW = NC * NS
BPW = BATCH // NW
LANES = 16
ROW = NUM_CTX * BPW + BPW


def _tc_body(side_ref, logit_ref, v_ref, b_ref, w_ref, bc_ref, buf_ref):
    proj = jnp.dot(v_ref[...], side_ref[...],
                   preferred_element_type=jnp.float32)
    bits = (proj > b_ref[...]).astype(jnp.float32)
    ctxf = jnp.sum(bits * bc_ref[...], axis=0)
    dots = jnp.dot(w_ref[...], logit_ref[...],
                   preferred_element_type=jnp.float32)
    wpb = BB // BPW
    merged = jnp.concatenate(
        [dots.reshape(NUM_CTX, wpb, BPW).swapaxes(0, 1).reshape(wpb, NUM_CTX * BPW),
         ctxf.reshape(wpb, BPW)], axis=1)
    buf_ref[...] = merged.reshape(wpb, 1, ROW)


def _sc_route(buf_hbm, out_hbm, buf_v, out_v):
    wid = lax.axis_index("s") * NC + lax.axis_index("c")
    base = wid * BPW
    pltpu.sync_copy(buf_hbm.at[wid, 0], buf_v)
    for i in range(BPW // LANES):
        rows = buf_v[pl.ds(NUM_CTX * BPW + i * LANES, LANES)].astype(jnp.int32)
        acc = jnp.zeros((LANES,), jnp.float32)
        for k in range(NUM_CTX):
            val = buf_v[pl.ds(k * BPW + i * LANES, LANES)]
            acc = jnp.where(rows == k, val, acc)
        out_v[pl.ds(i * LANES, LANES)] = acc
    pltpu.sync_copy(out_v, out_hbm.at[pl.ds(base, BPW)])


def kernel(logit_previous, side_information, v, b, weights, boolean_converter):
    grid = BATCH // BB
    buf = pl.pallas_call(
        _tc_body,
        grid=(grid,),
        in_specs=[
            pl.BlockSpec((SIDE_DIM, BB), lambda i: (0, i)),
            pl.BlockSpec((INPUT_DIM, BB), lambda i: (0, i)),
            pl.BlockSpec((CONTEXT_DIM, SIDE_DIM), lambda i: (0, 0)),
            pl.BlockSpec((CONTEXT_DIM, 1), lambda i: (0, 0)),
            pl.BlockSpec((NUM_CTX, INPUT_DIM), lambda i: (0, 0)),
            pl.BlockSpec((CONTEXT_DIM, 1), lambda i: (0, 0)),
        ],
        out_specs=pl.BlockSpec((BB // BPW, 1, ROW), lambda i: (i, 0, 0)),
        out_shape=jax.ShapeDtypeStruct((NW, 1, ROW), jnp.float32),
    )(side_information, logit_previous, v, b, weights, boolean_converter)

    route = functools.partial(
        pl.kernel,
        mesh=plsc.VectorSubcoreMesh(core_axis_name="c", subcore_axis_name="s"),
        out_type=jax.ShapeDtypeStruct((BATCH,), jnp.float32),
        scratch_types=[
            pltpu.VMEM((ROW,), jnp.float32),
            pltpu.VMEM((BPW,), jnp.float32),
        ],
    )(_sc_route)
    return route(buf)
